# baseline (speedup 1.0000x reference)
import functools

import jax
import jax.numpy as jnp
from jax import lax
from jax.experimental import pallas as pl
from jax.experimental.pallas import tpu as pltpu
from jax.experimental.pallas import tpu_sc as plsc

N = 10000
E = 80000
IN = 32
H = 32
EF = 16
EH = 64
G = 64
EPS = 1e-5

NW = 32
CH = 128
NCH = 20
PW = NCH * CH
EP = NW * PW
NP = 10240
ZR = NP // 16
BE = 2048
RB = BE // 4


def _sc_mesh():
    return plsc.VectorSubcoreMesh(core_axis_name="c", subcore_axis_name="s")


GC0 = 30
GC1 = 10


def _sc_gather(table, idxc):
    @functools.partial(
        pl.kernel, mesh=_sc_mesh(),
        out_type=jax.ShapeDtypeStruct((EP, 32), jnp.float32),
        compiler_params=pltpu.CompilerParams(use_tc_tiling_on_sc=False),
        scratch_types=[
            pltpu.VMEM((GC0, CH), jnp.int32),
            pltpu.VMEM((GC0 * CH, 32), jnp.float32),
            pltpu.SemaphoreType.DMA,
        ],
    )
    def k(table_hbm, idx_hbm, out_hbm, idx_v, rows_v, sem):
        c = lax.axis_index("c")
        s = lax.axis_index("s")

        def run(nch, chunk_base):
            pltpu.sync_copy(idx_hbm.at[pl.ds(chunk_base, nch)],
                            idx_v.at[pl.ds(0, nch)])
            cps = [
                pltpu.async_copy(table_hbm.at[idx_v.at[j]],
                                 rows_v.at[pl.ds(j * CH, CH)], sem)
                for j in range(nch)
            ]
            for cp in cps:
                cp.wait()
            pltpu.sync_copy(rows_v.at[pl.ds(0, nch * CH)],
                            out_hbm.at[pl.ds(chunk_base * CH, nch * CH)])

        @pl.when(c == 0)
        def _():
            run(GC0, s * GC0)

        @pl.when(c == 1)
        def _():
            run(GC1, 16 * GC0 + s * GC1)

    return k(table, idxc)


def _sc_scatter(msg, idx3, zinit):
    @functools.partial(
        pl.kernel, mesh=_sc_mesh(),
        out_type=jax.ShapeDtypeStruct((2, NP, 32), jnp.float32),
        compiler_params=pltpu.CompilerParams(use_tc_tiling_on_sc=False),
        scratch_types=[
            pltpu.VMEM((NCH, CH), jnp.int32),
            pltpu.VMEM((PW, 32), jnp.float32),
            pltpu.VMEM_SHARED((NP, 32), jnp.float32),
        ],
    )
    def k(msg_hbm, idx_hbm, z_hbm, out_hbm, idx_v, rows_v, acc_sh):
        c = lax.axis_index("c")
        s = lax.axis_index("s")
        wid = s * 2 + c
        pltpu.sync_copy(z_hbm.at[pl.ds(s * ZR, ZR)],
                        acc_sh.at[pl.ds(s * ZR, ZR)])
        plsc.subcore_barrier()
        pltpu.sync_copy(idx_hbm.at[wid], idx_v)
        pltpu.sync_copy(msg_hbm.at[pl.ds(wid * PW, PW)], rows_v)
        for j in range(NCH):
            pltpu.sync_copy(rows_v.at[pl.ds(j * CH, CH)],
                            acc_sh.at[idx_v.at[j]], add=True)
        plsc.subcore_barrier()
        pltpu.sync_copy(acc_sh.at[pl.ds(s * ZR, ZR)],
                        out_hbm.at[c, pl.ds(s * ZR, ZR)])

    return k(msg, idx3, zinit)


def _edge_body(ea_ref, g_ref, w1p_ref, be1p_ref, we2_ref, be2_ref, rep_ref,
               out_ref):
    ehp = jnp.maximum(
        jnp.dot(ea_ref[...], w1p_ref[...],
                preferred_element_type=jnp.float32) + be1p_ref[...], 0.0)
    g = g_ref[...]
    for q in range(4):
        t = jnp.dot(ehp[:, 64 * q:64 * q + 64], we2_ref[...],
                    preferred_element_type=jnp.float32) + be2_ref[...]
        grep = jnp.dot(g[:, 32 * q:32 * q + 32].astype(jnp.bfloat16),
                       rep_ref[...], preferred_element_type=jnp.float32)
        p = t.astype(jnp.bfloat16).astype(jnp.float32) * grep
        s = p[:, 0:128]
        for m in range(1, 8):
            s = s + p[:, m * 128:(m + 1) * 128]
        out_ref[:, 32 * q:32 * q + 32] = (
            s[:, 0:32] + s[:, 32:64] + s[:, 64:96] + s[:, 96:128])


def _tc_edge(eap, g, w1p, be1p, we2, be2, rep):
    return pl.pallas_call(
        _edge_body,
        grid=(EP // BE,),
        in_specs=[
            pl.BlockSpec((RB, 128), lambda i: (i, 0)),
            pl.BlockSpec((RB, 128), lambda i: (i, 0)),
            pl.BlockSpec((128, 4 * EH), lambda i: (0, 0)),
            pl.BlockSpec((1, 4 * EH), lambda i: (0, 0)),
            pl.BlockSpec((EH, 32 * H), lambda i: (0, 0)),
            pl.BlockSpec((1, 32 * H), lambda i: (0, 0)),
            pl.BlockSpec((32, 32 * H), lambda i: (0, 0)),
        ],
        out_specs=pl.BlockSpec((RB, 128), lambda i: (i, 0)),
        out_shape=jax.ShapeDtypeStruct((EP // 4, 128), jnp.float32),
    )(eap, g, w1p, be1p, we2, be2, rep)


def _node_body(parts_ref, h_ref, rootp_ref, biasp_ref, scalep_ref,
               shiftp_ref, out_ref):
    p = parts_ref[0] + parts_ref[1]
    t = p + jnp.dot(h_ref[...], rootp_ref[...],
                    preferred_element_type=jnp.float32) + biasp_ref[...]
    out_ref[...] = jnp.maximum(t * scalep_ref[...] + shiftp_ref[...], 0.0)


def _tc_node(parts, hp, rootp, biasp, scalep, shiftp):
    return pl.pallas_call(
        _node_body,
        out_shape=jax.ShapeDtypeStruct((NP // 4, 128), jnp.float32),
    )(parts, hp, rootp, biasp, scalep, shiftp)


def _pool_body(hv_ref, bv_ref, l1w_ref, l1b_ref, l2w_ref, l2b_ref, out_ref,
               pooled_ref):
    hv = hv_ref[...]
    bv = bv_ref[...]

    for gidx in range(G):
        v = jnp.where(bv == gidx, hv, -jnp.inf)
        r = jnp.max(v, axis=0, keepdims=True)
        r = jnp.maximum(jnp.maximum(r[:, 0:32], r[:, 32:64]),
                        jnp.maximum(r[:, 64:96], r[:, 96:128]))
        pooled_ref[pl.ds(gidx, 1), :] = r
    z = jnp.maximum(
        jnp.dot(pooled_ref[...], l1w_ref[...],
                preferred_element_type=jnp.float32) + l1b_ref[...], 0.0)
    out_ref[...] = jnp.dot(z, l2w_ref[...],
                           preferred_element_type=jnp.float32) + l2b_ref[...]


def _tc_pool(hview, belem, l1w, l1b, l2w, l2b):
    return pl.pallas_call(
        _pool_body,
        out_shape=jax.ShapeDtypeStruct((G, 2), jnp.float32),
        scratch_shapes=[pltpu.VMEM((G, H), jnp.float32)],
    )(hview, belem, l1w, l1b, l2w, l2b)


def _tile4(v):
    return jnp.tile(v.reshape(1, -1), (1, 4))


def kernel(x, edge_index, edge_attr, batch,
           We1_0, be1_0, We2_0, be2_0, root_0, bias_0, bng_0, bnb_0,
           We1_1, be1_1, We2_1, be2_1, root_1, bias_1, bng_1, bnb_1,
           lin1W, lin1b, lin2W, lin2b):
    src = edge_index[0].astype(jnp.int32)
    dst = edge_index[1].astype(jnp.int32)
    pad = EP - E
    srcp = jnp.concatenate([src, jnp.zeros((pad,), jnp.int32)]
                           ).reshape(EP // CH, CH)
    dstp = jnp.concatenate([dst, jnp.full((pad,), N, jnp.int32)]
                           ).reshape(NW, NCH, CH)
    eap = jnp.pad(edge_attr, ((0, pad), (0, 0))).reshape(EP // 4, 4 * EF)
    eap = jnp.pad(eap, ((0, 0), (0, 128 - 4 * EF)))
    zinit = jnp.zeros((NP, 32), jnp.float32)
    rep = jnp.kron(jnp.eye(32, dtype=jnp.float32),
                   jnp.ones((1, H), jnp.float32)).astype(jnp.bfloat16)
    eye4 = jnp.eye(4, dtype=jnp.float32)
    w1p_0 = jnp.pad(jnp.kron(eye4, We1_0), ((0, 64), (0, 0)))
    w1p_1 = jnp.pad(jnp.kron(eye4, We1_1), ((0, 64), (0, 0)))
    rootp_0 = jnp.kron(eye4, root_0)
    rootp_1 = jnp.kron(eye4, root_1)
    inv = 1.0 / jnp.sqrt(1.0 + EPS)
    xp = jnp.concatenate([x, jnp.zeros((NP - N, 32), jnp.float32)]
                         ).reshape(NP // 4, 128)

    g0 = _sc_gather(x, srcp).reshape(EP // 4, 128)
    msg0 = _tc_edge(eap, g0, w1p_0, _tile4(be1_0), We2_0,
                    be2_0.reshape(1, IN * H), rep)
    parts0 = _sc_scatter(msg0.reshape(EP, 32), dstp, zinit)
    h1 = _tc_node(parts0.reshape(2, NP // 4, 128), xp, rootp_0,
                  _tile4(bias_0), _tile4(bng_0 * inv), _tile4(bnb_0))

    g1 = _sc_gather(h1.reshape(NP, 32), srcp).reshape(EP // 4, 128)
    msg1 = _tc_edge(eap, g1, w1p_1, _tile4(be1_1), We2_1,
                    be2_1.reshape(1, H * H), rep)
    parts1 = _sc_scatter(msg1.reshape(EP, 32), dstp, zinit)
    h2 = _tc_node(parts1.reshape(2, NP // 4, 128), h1, rootp_1,
                  _tile4(bias_1), _tile4(bng_1 * inv), _tile4(bnb_1))

    belem = jnp.concatenate(
        [jnp.repeat(batch.astype(jnp.int32), H),
         jnp.full(((NP - N) * 32,), 2 ** 30, jnp.int32)]
    ).reshape(NP // 4, 128)
    return _tc_pool(h2, belem, lin1W, lin1b.reshape(1, H), lin2W,
                    lin2b.reshape(1, 2))


# speedup vs baseline: 1.0487x; 1.0487x over previous
import functools

import jax
import jax.numpy as jnp
from jax import lax
from jax.experimental import pallas as pl
from jax.experimental.pallas import tpu as pltpu
from jax.experimental.pallas import tpu_sc as plsc

N = 10000
E = 80000
IN = 32
H = 32
EF = 16
EH = 64
G = 64
EPS = 1e-5

NW = 32
CH = 128
NCH = 20
PW = NCH * CH
EP = NW * PW
NP = 10240
ZR = NP // 16
BE = 2048
RB = BE // 4


def _sc_mesh():
    return plsc.VectorSubcoreMesh(core_axis_name="c", subcore_axis_name="s")


GC0 = 20
GC1 = 20


def _sc_gather(table, idxc):
    @functools.partial(
        pl.kernel, mesh=_sc_mesh(),
        out_type=jax.ShapeDtypeStruct((EP, 32), jnp.float32),
        compiler_params=pltpu.CompilerParams(use_tc_tiling_on_sc=False),
        scratch_types=[
            pltpu.VMEM((GC0, CH), jnp.int32),
            pltpu.VMEM((GC0 * CH, 32), jnp.float32),
            pltpu.SemaphoreType.DMA,
        ],
    )
    def k(table_hbm, idx_hbm, out_hbm, idx_v, rows_v, sem):
        c = lax.axis_index("c")
        s = lax.axis_index("s")

        def run(nch, chunk_base):
            pltpu.sync_copy(idx_hbm.at[pl.ds(chunk_base, nch)],
                            idx_v.at[pl.ds(0, nch)])
            cps = [
                pltpu.async_copy(table_hbm.at[idx_v.at[j]],
                                 rows_v.at[pl.ds(j * CH, CH)], sem)
                for j in range(nch)
            ]
            for cp in cps:
                cp.wait()
            pltpu.sync_copy(rows_v.at[pl.ds(0, nch * CH)],
                            out_hbm.at[pl.ds(chunk_base * CH, nch * CH)])

        @pl.when(c == 0)
        def _():
            run(GC0, s * GC0)

        @pl.when(c == 1)
        def _():
            run(GC1, 16 * GC0 + s * GC1)

    return k(table, idxc)


def _sc_scatter(msg, idx3, zinit):
    @functools.partial(
        pl.kernel, mesh=_sc_mesh(),
        out_type=jax.ShapeDtypeStruct((2, NP, 32), jnp.float32),
        compiler_params=pltpu.CompilerParams(use_tc_tiling_on_sc=False),
        scratch_types=[
            pltpu.VMEM((NCH, CH), jnp.int32),
            pltpu.VMEM((PW, 32), jnp.float32),
            pltpu.VMEM_SHARED((NP, 32), jnp.float32),
        ],
    )
    def k(msg_hbm, idx_hbm, z_hbm, out_hbm, idx_v, rows_v, acc_sh):
        c = lax.axis_index("c")
        s = lax.axis_index("s")
        wid = s * 2 + c
        pltpu.sync_copy(z_hbm.at[pl.ds(s * ZR, ZR)],
                        acc_sh.at[pl.ds(s * ZR, ZR)])
        plsc.subcore_barrier()
        pltpu.sync_copy(idx_hbm.at[wid], idx_v)
        pltpu.sync_copy(msg_hbm.at[pl.ds(wid * PW, PW)], rows_v)
        for j in range(NCH):
            pltpu.sync_copy(rows_v.at[pl.ds(j * CH, CH)],
                            acc_sh.at[idx_v.at[j]], add=True)
        plsc.subcore_barrier()
        pltpu.sync_copy(acc_sh.at[pl.ds(s * ZR, ZR)],
                        out_hbm.at[c, pl.ds(s * ZR, ZR)])

    return k(msg, idx3, zinit)


def _edge_body(ea_ref, g_ref, w1p_ref, be1p_ref, we2_ref, be2_ref, rep_ref,
               out_ref):
    ehp = jnp.maximum(
        jnp.dot(ea_ref[...], w1p_ref[...],
                preferred_element_type=jnp.float32) + be1p_ref[...], 0.0)
    g = g_ref[...]
    for q in range(4):
        t = jnp.dot(ehp[:, 64 * q:64 * q + 64], we2_ref[...],
                    preferred_element_type=jnp.float32) + be2_ref[...]
        grep = jnp.dot(g[:, 32 * q:32 * q + 32].astype(jnp.bfloat16),
                       rep_ref[...], preferred_element_type=jnp.float32)
        p = t.astype(jnp.bfloat16).astype(jnp.float32) * grep
        s = p[:, 0:128]
        for m in range(1, 8):
            s = s + p[:, m * 128:(m + 1) * 128]
        out_ref[:, 32 * q:32 * q + 32] = (
            s[:, 0:32] + s[:, 32:64] + s[:, 64:96] + s[:, 96:128])


def _tc_edge(eap, g, w1p, be1p, we2, be2, rep):
    return pl.pallas_call(
        _edge_body,
        grid=(EP // BE,),
        in_specs=[
            pl.BlockSpec((RB, 4 * EF), lambda i: (i, 0)),
            pl.BlockSpec((RB, 128), lambda i: (i, 0)),
            pl.BlockSpec((4 * EF, 4 * EH), lambda i: (0, 0)),
            pl.BlockSpec((1, 4 * EH), lambda i: (0, 0)),
            pl.BlockSpec((EH, 32 * H), lambda i: (0, 0)),
            pl.BlockSpec((1, 32 * H), lambda i: (0, 0)),
            pl.BlockSpec((32, 32 * H), lambda i: (0, 0)),
        ],
        out_specs=pl.BlockSpec((RB, 128), lambda i: (i, 0)),
        out_shape=jax.ShapeDtypeStruct((EP // 4, 128), jnp.float32),
    )(eap, g, w1p, be1p, we2, be2, rep)


def _node_body(parts_ref, h_ref, rootp_ref, biasp_ref, scalep_ref,
               shiftp_ref, out_ref):
    p = parts_ref[0] + parts_ref[1]
    t = p + jnp.dot(h_ref[...], rootp_ref[...],
                    preferred_element_type=jnp.float32) + biasp_ref[...]
    out_ref[...] = jnp.maximum(t * scalep_ref[...] + shiftp_ref[...], 0.0)


def _tc_node(parts, hp, rootp, biasp, scalep, shiftp):
    return pl.pallas_call(
        _node_body,
        out_shape=jax.ShapeDtypeStruct((NP // 4, 128), jnp.float32),
    )(parts, hp, rootp, biasp, scalep, shiftp)


def _pool_body(hv_ref, bv_ref, l1w_ref, l1b_ref, l2w_ref, l2b_ref, out_ref,
               pooled_ref):
    hv = hv_ref[...]
    bv = bv_ref[...]

    for gidx in range(G):
        v = jnp.where(bv == gidx, hv, -jnp.inf)
        r = jnp.max(v, axis=0, keepdims=True)
        r = jnp.maximum(jnp.maximum(r[:, 0:32], r[:, 32:64]),
                        jnp.maximum(r[:, 64:96], r[:, 96:128]))
        pooled_ref[pl.ds(gidx, 1), :] = r
    z = jnp.maximum(
        jnp.dot(pooled_ref[...], l1w_ref[...],
                preferred_element_type=jnp.float32) + l1b_ref[...], 0.0)
    out_ref[...] = jnp.dot(z, l2w_ref[...],
                           preferred_element_type=jnp.float32) + l2b_ref[...]


def _tc_pool(hview, belem, l1w, l1b, l2w, l2b):
    return pl.pallas_call(
        _pool_body,
        out_shape=jax.ShapeDtypeStruct((G, 2), jnp.float32),
        scratch_shapes=[pltpu.VMEM((G, H), jnp.float32)],
    )(hview, belem, l1w, l1b, l2w, l2b)


def _tile4(v):
    return jnp.tile(v.reshape(1, -1), (1, 4))


def kernel(x, edge_index, edge_attr, batch,
           We1_0, be1_0, We2_0, be2_0, root_0, bias_0, bng_0, bnb_0,
           We1_1, be1_1, We2_1, be2_1, root_1, bias_1, bng_1, bnb_1,
           lin1W, lin1b, lin2W, lin2b):
    src = edge_index[0].astype(jnp.int32)
    dst = edge_index[1].astype(jnp.int32)
    pad = EP - E
    srcp = jnp.concatenate([src, jnp.zeros((pad,), jnp.int32)]
                           ).reshape(EP // CH, CH)
    dstp = jnp.concatenate([dst, jnp.full((pad,), N, jnp.int32)]
                           ).reshape(NW, NCH, CH)
    eap = jnp.pad(edge_attr, ((0, pad), (0, 0))).reshape(EP // 4, 4 * EF)
    zinit = jnp.zeros((NP, 32), jnp.float32)
    rep = jnp.kron(jnp.eye(32, dtype=jnp.float32),
                   jnp.ones((1, H), jnp.float32)).astype(jnp.bfloat16)
    eye4 = jnp.eye(4, dtype=jnp.float32)
    w1p_0 = jnp.kron(eye4, We1_0)
    w1p_1 = jnp.kron(eye4, We1_1)
    rootp_0 = jnp.kron(eye4, root_0)
    rootp_1 = jnp.kron(eye4, root_1)
    inv = 1.0 / jnp.sqrt(1.0 + EPS)
    xp = jnp.concatenate([x, jnp.zeros((NP - N, 32), jnp.float32)]
                         ).reshape(NP // 4, 128)

    g0 = _sc_gather(x, srcp).reshape(EP // 4, 128)
    msg0 = _tc_edge(eap, g0, w1p_0, _tile4(be1_0), We2_0,
                    be2_0.reshape(1, IN * H), rep)
    parts0 = _sc_scatter(msg0.reshape(EP, 32), dstp, zinit)
    h1 = _tc_node(parts0.reshape(2, NP // 4, 128), xp, rootp_0,
                  _tile4(bias_0), _tile4(bng_0 * inv), _tile4(bnb_0))

    g1 = _sc_gather(h1.reshape(NP, 32), srcp).reshape(EP // 4, 128)
    msg1 = _tc_edge(eap, g1, w1p_1, _tile4(be1_1), We2_1,
                    be2_1.reshape(1, H * H), rep)
    parts1 = _sc_scatter(msg1.reshape(EP, 32), dstp, zinit)
    h2 = _tc_node(parts1.reshape(2, NP // 4, 128), h1, rootp_1,
                  _tile4(bias_1), _tile4(bng_1 * inv), _tile4(bnb_1))

    belem = jnp.concatenate(
        [jnp.repeat(batch.astype(jnp.int32), H),
         jnp.full(((NP - N) * 32,), 2 ** 30, jnp.int32)]
    ).reshape(NP // 4, 128)
    return _tc_pool(h2, belem, lin1W, lin1b.reshape(1, H), lin2W,
                    lin2b.reshape(1, 2))


# speedup vs baseline: 1.1081x; 1.0567x over previous
import functools

import jax
import jax.numpy as jnp
from jax import lax
from jax.experimental import pallas as pl
from jax.experimental.pallas import tpu as pltpu
from jax.experimental.pallas import tpu_sc as plsc

N = 10000
E = 80000
IN = 32
H = 32
EF = 16
EH = 64
G = 64
EPS = 1e-5

NW = 32
CH = 128
EP = 81920
EH2 = EP // 2
NCH = EH2 // (NW * CH)
PW = NCH * CH
NP = 10240
ZR = NP // 16
BE = 2048
RB = BE // 4
NBLK = EH2 // BE


def _sc_mesh():
    return plsc.VectorSubcoreMesh(core_axis_name="c", subcore_axis_name="s")


def _sc_gather(table, idxc):
    @functools.partial(
        pl.kernel, mesh=_sc_mesh(),
        out_type=jax.ShapeDtypeStruct((EH2, 32), jnp.float32),
        compiler_params=pltpu.CompilerParams(use_tc_tiling_on_sc=False),
        scratch_types=[
            pltpu.VMEM((NCH, CH), jnp.int32),
            pltpu.VMEM((PW, 32), jnp.float32),
            pltpu.SemaphoreType.DMA,
        ],
    )
    def k(table_hbm, idx_hbm, out_hbm, idx_v, rows_v, sem):
        wid = lax.axis_index("s") * 2 + lax.axis_index("c")
        pltpu.sync_copy(idx_hbm.at[pl.ds(wid * NCH, NCH)], idx_v)
        cps = [
            pltpu.async_copy(table_hbm.at[idx_v.at[j]],
                             rows_v.at[pl.ds(j * CH, CH)], sem)
            for j in range(NCH)
        ]
        for cp in cps:
            cp.wait()
        pltpu.sync_copy(rows_v, out_hbm.at[pl.ds(wid * PW, PW)])

    return k(table, idxc)


def _sc_scatter(msg, idx3, zinit):
    @functools.partial(
        pl.kernel, mesh=_sc_mesh(),
        out_type=jax.ShapeDtypeStruct((2, NP, 32), jnp.float32),
        compiler_params=pltpu.CompilerParams(use_tc_tiling_on_sc=False),
        scratch_types=[
            pltpu.VMEM((NCH, CH), jnp.int32),
            pltpu.VMEM((PW, 32), jnp.float32),
            pltpu.VMEM_SHARED((NP, 32), jnp.float32),
        ],
    )
    def k(msg_hbm, idx_hbm, z_hbm, out_hbm, idx_v, rows_v, acc_sh):
        c = lax.axis_index("c")
        s = lax.axis_index("s")
        wid = s * 2 + c
        pltpu.sync_copy(z_hbm.at[pl.ds(s * ZR, ZR)],
                        acc_sh.at[pl.ds(s * ZR, ZR)])
        plsc.subcore_barrier()
        pltpu.sync_copy(idx_hbm.at[wid], idx_v)
        pltpu.sync_copy(msg_hbm.at[pl.ds(wid * PW, PW)], rows_v)
        for j in range(NCH):
            pltpu.sync_copy(rows_v.at[pl.ds(j * CH, CH)],
                            acc_sh.at[idx_v.at[j]], add=True)
        plsc.subcore_barrier()
        pltpu.sync_copy(acc_sh.at[pl.ds(s * ZR, ZR)],
                        out_hbm.at[c, pl.ds(s * ZR, ZR)])

    return k(msg, idx3, zinit)


def _edge_body(ea_ref, g_ref, w1p_ref, be1p_ref, we2_ref, be2_ref, rep_ref,
               out_ref):
    ehp = jnp.maximum(
        jnp.dot(ea_ref[...], w1p_ref[...],
                preferred_element_type=jnp.float32) + be1p_ref[...], 0.0)
    g = g_ref[...]
    for q in range(4):
        t = jnp.dot(ehp[:, 64 * q:64 * q + 64], we2_ref[...],
                    preferred_element_type=jnp.float32) + be2_ref[...]
        grep = jnp.dot(g[:, 32 * q:32 * q + 32].astype(jnp.bfloat16),
                       rep_ref[...], preferred_element_type=jnp.float32)
        p = t.astype(jnp.bfloat16).astype(jnp.float32) * grep
        s = p[:, 0:128]
        for m in range(1, 8):
            s = s + p[:, m * 128:(m + 1) * 128]
        out_ref[:, 32 * q:32 * q + 32] = (
            s[:, 0:32] + s[:, 32:64] + s[:, 64:96] + s[:, 96:128])


def _tc_edge(eap, g, w1p, be1p, we2, be2, rep, off):
    return pl.pallas_call(
        _edge_body,
        grid=(NBLK,),
        in_specs=[
            pl.BlockSpec((RB, 4 * EF), lambda i: (i + off, 0)),
            pl.BlockSpec((RB, 128), lambda i: (i, 0)),
            pl.BlockSpec((4 * EF, 4 * EH), lambda i: (0, 0)),
            pl.BlockSpec((1, 4 * EH), lambda i: (0, 0)),
            pl.BlockSpec((EH, 32 * H), lambda i: (0, 0)),
            pl.BlockSpec((1, 32 * H), lambda i: (0, 0)),
            pl.BlockSpec((32, 32 * H), lambda i: (0, 0)),
        ],
        out_specs=pl.BlockSpec((RB, 128), lambda i: (i, 0)),
        out_shape=jax.ShapeDtypeStruct((EH2 // 4, 128), jnp.float32),
    )(eap, g, w1p, be1p, we2, be2, rep)


def _node_body(pa_ref, pb_ref, h_ref, rootp_ref, biasp_ref, scalep_ref,
               shiftp_ref, out_ref):
    p = (pa_ref[0] + pa_ref[1]) + (pb_ref[0] + pb_ref[1])
    t = p + jnp.dot(h_ref[...], rootp_ref[...],
                    preferred_element_type=jnp.float32) + biasp_ref[...]
    out_ref[...] = jnp.maximum(t * scalep_ref[...] + shiftp_ref[...], 0.0)


def _tc_node(pa, pb, hp, rootp, biasp, scalep, shiftp):
    return pl.pallas_call(
        _node_body,
        out_shape=jax.ShapeDtypeStruct((NP // 4, 128), jnp.float32),
    )(pa, pb, hp, rootp, biasp, scalep, shiftp)


def _pool_body(hv_ref, bv_ref, l1w_ref, l1b_ref, l2w_ref, l2b_ref, out_ref,
               pooled_ref):
    hv = hv_ref[...]
    bv = bv_ref[...]

    for gidx in range(G):
        v = jnp.where(bv == gidx, hv, -jnp.inf)
        r = jnp.max(v, axis=0, keepdims=True)
        r = jnp.maximum(jnp.maximum(r[:, 0:32], r[:, 32:64]),
                        jnp.maximum(r[:, 64:96], r[:, 96:128]))
        pooled_ref[pl.ds(gidx, 1), :] = r
    z = jnp.maximum(
        jnp.dot(pooled_ref[...], l1w_ref[...],
                preferred_element_type=jnp.float32) + l1b_ref[...], 0.0)
    out_ref[...] = jnp.dot(z, l2w_ref[...],
                           preferred_element_type=jnp.float32) + l2b_ref[...]


def _tc_pool(hview, belem, l1w, l1b, l2w, l2b):
    return pl.pallas_call(
        _pool_body,
        out_shape=jax.ShapeDtypeStruct((G, 2), jnp.float32),
        scratch_shapes=[pltpu.VMEM((G, H), jnp.float32)],
    )(hview, belem, l1w, l1b, l2w, l2b)


def _tile4(v):
    return jnp.tile(v.reshape(1, -1), (1, 4))


def kernel(x, edge_index, edge_attr, batch,
           We1_0, be1_0, We2_0, be2_0, root_0, bias_0, bng_0, bnb_0,
           We1_1, be1_1, We2_1, be2_1, root_1, bias_1, bng_1, bnb_1,
           lin1W, lin1b, lin2W, lin2b):
    src = edge_index[0].astype(jnp.int32)
    dst = edge_index[1].astype(jnp.int32)
    pad = EP - E
    srcp = jnp.concatenate([src, jnp.zeros((pad,), jnp.int32)])
    srcA = srcp[:EH2].reshape(EH2 // CH, CH)
    srcB = srcp[EH2:].reshape(EH2 // CH, CH)
    dstp = jnp.concatenate([dst, jnp.full((pad,), N, jnp.int32)])
    dstA = dstp[:EH2].reshape(NW, NCH, CH)
    dstB = dstp[EH2:].reshape(NW, NCH, CH)
    eap = jnp.pad(edge_attr, ((0, pad), (0, 0))).reshape(EP // 4, 4 * EF)
    zinit = jnp.zeros((NP, 32), jnp.float32)
    rep = jnp.kron(jnp.eye(32, dtype=jnp.float32),
                   jnp.ones((1, H), jnp.float32)).astype(jnp.bfloat16)
    eye4 = jnp.eye(4, dtype=jnp.float32)
    w1p_0 = jnp.kron(eye4, We1_0)
    w1p_1 = jnp.kron(eye4, We1_1)
    rootp_0 = jnp.kron(eye4, root_0)
    rootp_1 = jnp.kron(eye4, root_1)
    inv = 1.0 / jnp.sqrt(1.0 + EPS)
    xp = jnp.concatenate([x, jnp.zeros((NP - N, 32), jnp.float32)]
                         ).reshape(NP // 4, 128)

    def layer(table, hp, w1p, be1, we2, be2, rootp, bias, scale, shift):
        gA = _sc_gather(table, srcA).reshape(EH2 // 4, 128)
        gB = _sc_gather(table, srcB).reshape(EH2 // 4, 128)
        msgA = _tc_edge(eap, gA, w1p, _tile4(be1), we2, be2, rep, 0)
        msgB = _tc_edge(eap, gB, w1p, _tile4(be1), we2, be2, rep, NBLK)
        pA = _sc_scatter(msgA.reshape(EH2, 32), dstA, zinit)
        pB = _sc_scatter(msgB.reshape(EH2, 32), dstB, zinit)
        return _tc_node(pA.reshape(2, NP // 4, 128),
                        pB.reshape(2, NP // 4, 128), hp, rootp,
                        _tile4(bias), _tile4(scale), _tile4(shift))

    h1 = layer(x, xp, w1p_0, be1_0, We2_0, be2_0.reshape(1, IN * H),
               rootp_0, bias_0, bng_0 * inv, bnb_0)
    h2 = layer(h1.reshape(NP, 32), h1, w1p_1, be1_1, We2_1,
               be2_1.reshape(1, H * H), rootp_1, bias_1, bng_1 * inv, bnb_1)

    belem = jnp.concatenate(
        [jnp.repeat(batch.astype(jnp.int32), H),
         jnp.full(((NP - N) * 32,), 2 ** 30, jnp.int32)]
    ).reshape(NP // 4, 128)
    return _tc_pool(h2, belem, lin1W, lin1b.reshape(1, H), lin2W,
                    lin2b.reshape(1, 2))


# speedup vs baseline: 1.1715x; 1.0572x over previous
import functools

import jax
import jax.numpy as jnp
from jax import lax
from jax.experimental import pallas as pl
from jax.experimental.pallas import tpu as pltpu
from jax.experimental.pallas import tpu_sc as plsc

N = 10000
E = 80000
IN = 32
H = 32
EF = 16
EH = 64
G = 64
EPS = 1e-5

NW = 32
CH = 128
EP = 81920
EH2 = EP // 2
NCH = EH2 // (NW * CH)
PW = NCH * CH
NP = 10240
ZR = NP // 16
BE = 2048
RB = BE // 4
NBLK = EH2 // BE


def _sc_mesh():
    return plsc.VectorSubcoreMesh(core_axis_name="c", subcore_axis_name="s")


def _sc_gather(table, idxc):
    @functools.partial(
        pl.kernel, mesh=_sc_mesh(),
        out_type=jax.ShapeDtypeStruct((EH2, 32), jnp.float32),
        compiler_params=pltpu.CompilerParams(use_tc_tiling_on_sc=False),
        scratch_types=[
            pltpu.VMEM((NCH, CH), jnp.int32),
            pltpu.VMEM((PW, 32), jnp.float32),
            pltpu.SemaphoreType.DMA,
        ],
    )
    def k(table_hbm, idx_hbm, out_hbm, idx_v, rows_v, sem):
        wid = lax.axis_index("s") * 2 + lax.axis_index("c")
        pltpu.sync_copy(idx_hbm.at[pl.ds(wid * NCH, NCH)], idx_v)
        cps = [
            pltpu.async_copy(table_hbm.at[idx_v.at[j]],
                             rows_v.at[pl.ds(j * CH, CH)], sem)
            for j in range(NCH)
        ]
        for cp in cps:
            cp.wait()
        pltpu.sync_copy(rows_v, out_hbm.at[pl.ds(wid * PW, PW)])

    return k(table, idxc)


def _sc_scatter(msg, idx3, zinit):
    @functools.partial(
        pl.kernel, mesh=_sc_mesh(),
        out_type=jax.ShapeDtypeStruct((2, NP, 32), jnp.float32),
        compiler_params=pltpu.CompilerParams(use_tc_tiling_on_sc=False),
        scratch_types=[
            pltpu.VMEM((NCH, CH), jnp.int32),
            pltpu.VMEM((PW, 32), jnp.float32),
            pltpu.VMEM_SHARED((NP, 32), jnp.float32),
        ],
    )
    def k(msg_hbm, idx_hbm, z_hbm, out_hbm, idx_v, rows_v, acc_sh):
        c = lax.axis_index("c")
        s = lax.axis_index("s")
        wid = s * 2 + c
        pltpu.sync_copy(z_hbm.at[pl.ds(s * ZR, ZR)],
                        acc_sh.at[pl.ds(s * ZR, ZR)])
        plsc.subcore_barrier()
        pltpu.sync_copy(idx_hbm.at[wid], idx_v)
        pltpu.sync_copy(msg_hbm.at[pl.ds(wid * PW, PW)], rows_v)
        for j in range(NCH):
            pltpu.sync_copy(rows_v.at[pl.ds(j * CH, CH)],
                            acc_sh.at[idx_v.at[j]], add=True)
        plsc.subcore_barrier()
        pltpu.sync_copy(acc_sh.at[pl.ds(s * ZR, ZR)],
                        out_hbm.at[c, pl.ds(s * ZR, ZR)])

    return k(msg, idx3, zinit)


def _edge_body(ea_ref, g_ref, w1p_ref, be1p_ref, we2_ref, be2_ref, rep_ref,
               out_ref):
    ehp = jnp.maximum(
        jnp.dot(ea_ref[...], w1p_ref[...],
                preferred_element_type=jnp.float32) + be1p_ref[...], 0.0)
    g = g_ref[...]
    for q in range(4):
        t = jnp.dot(ehp[:, 64 * q:64 * q + 64], we2_ref[...],
                    preferred_element_type=jnp.float32) + be2_ref[...]
        grep = jnp.dot(g[:, 32 * q:32 * q + 32].astype(jnp.bfloat16),
                       rep_ref[...], preferred_element_type=jnp.float32)
        p = t.astype(jnp.bfloat16).astype(jnp.float32) * grep
        s = p[:, 0:128]
        for m in range(1, 8):
            s = s + p[:, m * 128:(m + 1) * 128]
        out_ref[:, 32 * q:32 * q + 32] = (
            s[:, 0:32] + s[:, 32:64] + s[:, 64:96] + s[:, 96:128])


def _tc_edge(eap, g, w1p, be1p, we2, be2, rep, off):
    return pl.pallas_call(
        _edge_body,
        grid=(NBLK,),
        in_specs=[
            pl.BlockSpec((RB, 4 * EF), lambda i: (i + off, 0)),
            pl.BlockSpec((RB, 128), lambda i: (i, 0)),
            pl.BlockSpec((4 * EF, 4 * EH), lambda i: (0, 0)),
            pl.BlockSpec((1, 4 * EH), lambda i: (0, 0)),
            pl.BlockSpec((EH, 32 * H), lambda i: (0, 0)),
            pl.BlockSpec((1, 32 * H), lambda i: (0, 0)),
            pl.BlockSpec((32, 32 * H), lambda i: (0, 0)),
        ],
        out_specs=pl.BlockSpec((RB, 128), lambda i: (i, 0)),
        out_shape=jax.ShapeDtypeStruct((EH2 // 4, 128), jnp.float32),
    )(eap, g, w1p, be1p, we2, be2, rep)


def _node_body(pa_ref, pb_ref, h_ref, rootp_ref, biasp_ref, scalep_ref,
               shiftp_ref, out_ref):
    p = (pa_ref[0] + pa_ref[1]) + (pb_ref[0] + pb_ref[1])
    t = p + jnp.dot(h_ref[...], rootp_ref[...],
                    preferred_element_type=jnp.float32) + biasp_ref[...]
    out_ref[...] = jnp.maximum(t * scalep_ref[...] + shiftp_ref[...], 0.0)


def _tc_node(pa, pb, hp, rootp, biasp, scalep, shiftp):
    return pl.pallas_call(
        _node_body,
        out_shape=jax.ShapeDtypeStruct((NP // 4, 128), jnp.float32),
    )(pa, pb, hp, rootp, biasp, scalep, shiftp)


def _pool_body(hv_ref, bv_ref, l1w_ref, l1b_ref, l2w_ref, l2b_ref, out_ref,
               pooled_ref):
    hv = hv_ref[...]
    bv = bv_ref[...]

    for gidx in range(G):
        v = jnp.where(bv == gidx, hv, -jnp.inf)
        r = jnp.max(v, axis=0, keepdims=True)
        r = jnp.maximum(jnp.maximum(r[:, 0:32], r[:, 32:64]),
                        jnp.maximum(r[:, 64:96], r[:, 96:128]))
        pooled_ref[pl.ds(gidx, 1), :] = r
    z = jnp.maximum(
        jnp.dot(pooled_ref[...], l1w_ref[...],
                preferred_element_type=jnp.float32) + l1b_ref[...], 0.0)
    out_ref[...] = jnp.dot(z, l2w_ref[...],
                           preferred_element_type=jnp.float32) + l2b_ref[...]


def _tc_pool(hview, belem, l1w, l1b, l2w, l2b):
    return pl.pallas_call(
        _pool_body,
        out_shape=jax.ShapeDtypeStruct((G, 2), jnp.float32),
        scratch_shapes=[pltpu.VMEM((G, H), jnp.float32)],
    )(hview, belem, l1w, l1b, l2w, l2b)


def _tile4(v):
    return jnp.tile(v.reshape(1, -1), (1, 4))


def kernel(x, edge_index, edge_attr, batch,
           We1_0, be1_0, We2_0, be2_0, root_0, bias_0, bng_0, bnb_0,
           We1_1, be1_1, We2_1, be2_1, root_1, bias_1, bng_1, bnb_1,
           lin1W, lin1b, lin2W, lin2b):
    src = edge_index[0].astype(jnp.int32)
    dst = edge_index[1].astype(jnp.int32)
    pad = EP - E
    srcp = jnp.concatenate([src, jnp.zeros((pad,), jnp.int32)])
    srcA = srcp[:EH2].reshape(EH2 // CH, CH)
    srcB = srcp[EH2:].reshape(EH2 // CH, CH)
    dstp = jnp.concatenate([dst, jnp.full((pad,), N, jnp.int32)])
    dstA = dstp[:EH2].reshape(NW, NCH, CH)
    dstB = dstp[EH2:].reshape(NW, NCH, CH)
    eap = jnp.pad(edge_attr.reshape(E // 4, 4 * EF),
                  ((0, pad // 4), (0, 0)))
    zinit = jnp.zeros((NP, 32), jnp.float32)
    rep = jnp.kron(jnp.eye(32, dtype=jnp.float32),
                   jnp.ones((1, H), jnp.float32)).astype(jnp.bfloat16)
    eye4 = jnp.eye(4, dtype=jnp.float32)
    w1p_0 = jnp.kron(eye4, We1_0)
    w1p_1 = jnp.kron(eye4, We1_1)
    rootp_0 = jnp.kron(eye4, root_0)
    rootp_1 = jnp.kron(eye4, root_1)
    inv = 1.0 / jnp.sqrt(1.0 + EPS)
    xp = jnp.concatenate([x, jnp.zeros((NP - N, 32), jnp.float32)]
                         ).reshape(NP // 4, 128)

    def layer(table, hp, w1p, be1, we2, be2, rootp, bias, scale, shift):
        gA = _sc_gather(table, srcA).reshape(EH2 // 4, 128)
        gB = _sc_gather(table, srcB).reshape(EH2 // 4, 128)
        msgA = _tc_edge(eap, gA, w1p, _tile4(be1), we2, be2, rep, 0)
        msgB = _tc_edge(eap, gB, w1p, _tile4(be1), we2, be2, rep, NBLK)
        pA = _sc_scatter(msgA.reshape(EH2, 32), dstA, zinit)
        pB = _sc_scatter(msgB.reshape(EH2, 32), dstB, zinit)
        return _tc_node(pA.reshape(2, NP // 4, 128),
                        pB.reshape(2, NP // 4, 128), hp, rootp,
                        _tile4(bias), _tile4(scale), _tile4(shift))

    h1 = layer(x, xp, w1p_0, be1_0, We2_0, be2_0.reshape(1, IN * H),
               rootp_0, bias_0, bng_0 * inv, bnb_0)
    h2 = layer(h1.reshape(NP, 32), h1, w1p_1, be1_1, We2_1,
               be2_1.reshape(1, H * H), rootp_1, bias_1, bng_1 * inv, bnb_1)

    belem = jnp.concatenate(
        [jnp.repeat(batch.astype(jnp.int32), H),
         jnp.full(((NP - N) * 32,), 2 ** 30, jnp.int32)]
    ).reshape(NP // 4, 128)
    return _tc_pool(h2, belem, lin1W, lin1b.reshape(1, H), lin2W,
                    lin2b.reshape(1, 2))
